# baseline (speedup 1.0000x reference)
import jax
import jax.numpy as jnp
from jax.experimental import pallas as pl
from jax.experimental.pallas import tpu as pltpu
from jax.experimental.pallas import tpu_sc as plsc

_NUM_MODALITIES = 5
_EMBED_DIM = 1024
_NUM_SC = 2
_HALF = _EMBED_DIM // _NUM_SC


def kernel(modality_embeddings, modality_idx):
    idx = jnp.full((16,), modality_idx, dtype=jnp.int32)
    emb = modality_embeddings.reshape(_NUM_MODALITIES, _NUM_SC, _HALF)

    @pl.kernel(
        out_type=jax.ShapeDtypeStruct((_NUM_SC, _HALF),
                                      modality_embeddings.dtype),
        mesh=plsc.ScalarSubcoreMesh(axis_name="core", num_cores=_NUM_SC),
        scratch_types=[
            pltpu.SMEM((16,), jnp.int32),
            pltpu.SemaphoreType.DMA,
        ],
    )
    def _sc_lookup(emb_hbm, idx_hbm, out_hbm, idx_smem, sem):
        core = jax.lax.axis_index("core")
        pltpu.async_copy(idx_hbm, idx_smem, sem).wait()
        i = idx_smem[0]
        pltpu.async_copy(emb_hbm.at[i, core], out_hbm.at[core], sem).wait()

    return _sc_lookup(emb, idx).reshape(_EMBED_DIM)


# speedup vs baseline: 1.0254x; 1.0254x over previous
import jax
import jax.numpy as jnp
from jax.experimental import pallas as pl
from jax.experimental.pallas import tpu as pltpu
from jax.experimental.pallas import tpu_sc as plsc

_NUM_MODALITIES = 5
_EMBED_DIM = 1024
_NUM_SC = 2
_HALF = _EMBED_DIM // _NUM_SC


def kernel(modality_embeddings, modality_idx):
    idx = jnp.asarray(modality_idx, dtype=jnp.int32).reshape((1,))
    emb = modality_embeddings.reshape(_NUM_MODALITIES, _NUM_SC, _HALF)

    @pl.kernel(
        out_type=jax.ShapeDtypeStruct((_NUM_SC, _HALF),
                                      modality_embeddings.dtype),
        mesh=plsc.ScalarSubcoreMesh(axis_name="core", num_cores=_NUM_SC),
        scratch_types=[
            pltpu.SMEM((1,), jnp.int32),
            pltpu.SemaphoreType.DMA,
        ],
    )
    def _sc_lookup(emb_hbm, idx_hbm, out_hbm, idx_smem, sem):
        core = jax.lax.axis_index("core")
        pltpu.async_copy(idx_hbm, idx_smem, sem).wait()
        i = idx_smem[0]
        pltpu.async_copy(emb_hbm.at[i, core], out_hbm.at[core], sem).wait()

    return _sc_lookup(emb, idx).reshape(_EMBED_DIM)


# speedup vs baseline: 1.1899x; 1.1605x over previous
import jax
import jax.numpy as jnp
from jax.experimental import pallas as pl
from jax.experimental.pallas import tpu as pltpu
from jax.experimental.pallas import tpu_sc as plsc

_NUM_MODALITIES = 5
_EMBED_DIM = 1024
_NUM_SC = 1
_HALF = _EMBED_DIM // _NUM_SC


def kernel(modality_embeddings, modality_idx):
    idx = jnp.asarray(modality_idx, dtype=jnp.int32).reshape((1,))
    emb = modality_embeddings.reshape(_NUM_MODALITIES, _NUM_SC, _HALF)

    @pl.kernel(
        out_type=jax.ShapeDtypeStruct((_NUM_SC, _HALF),
                                      modality_embeddings.dtype),
        mesh=plsc.ScalarSubcoreMesh(axis_name="core", num_cores=_NUM_SC),
        scratch_types=[
            pltpu.SMEM((1,), jnp.int32),
            pltpu.SemaphoreType.DMA,
        ],
    )
    def _sc_lookup(emb_hbm, idx_hbm, out_hbm, idx_smem, sem):
        core = jax.lax.axis_index("core")
        pltpu.async_copy(idx_hbm, idx_smem, sem).wait()
        i = idx_smem[0]
        pltpu.async_copy(emb_hbm.at[i, core], out_hbm.at[core], sem).wait()

    return _sc_lookup(emb, idx).reshape(_EMBED_DIM)


# speedup vs baseline: 6.2046x; 5.2144x over previous
import jax
import jax.numpy as jnp
from jax.experimental import pallas as pl
from jax.experimental.pallas import tpu as pltpu

_NUM_MODALITIES = 5
_EMBED_DIM = 1024


def _copy_row(idx_ref, row_ref, out_ref):
    out_ref[...] = row_ref[...]


def kernel(modality_embeddings, modality_idx):
    idx = jnp.asarray(modality_idx, dtype=jnp.int32).reshape((1,))
    emb = modality_embeddings.reshape(_NUM_MODALITIES, 8, _EMBED_DIM // 8)
    out = pl.pallas_call(
        _copy_row,
        grid_spec=pltpu.PrefetchScalarGridSpec(
            num_scalar_prefetch=1,
            grid=(1,),
            in_specs=[
                pl.BlockSpec((1, 8, _EMBED_DIM // 8),
                             lambda i, idx_ref: (idx_ref[0], 0, 0)),
            ],
            out_specs=pl.BlockSpec((1, 8, _EMBED_DIM // 8),
                                   lambda i, idx_ref: (0, 0, 0)),
        ),
        out_shape=jax.ShapeDtypeStruct((1, 8, _EMBED_DIM // 8),
                                       modality_embeddings.dtype),
    )(idx, emb)
    return out.reshape(_EMBED_DIM)


# speedup vs baseline: 10.9232x; 1.7605x over previous
import jax
import jax.numpy as jnp
from jax.experimental import pallas as pl
from jax.experimental.pallas import tpu as pltpu

_NUM_MODALITIES = 5
_EMBED_DIM = 1024


def _row_dma(idx_ref, emb_hbm, out_hbm, sem):
    i = idx_ref[0]
    pltpu.make_async_copy(emb_hbm.at[i], out_hbm, sem).start()
    pltpu.make_async_copy(emb_hbm.at[i], out_hbm, sem).wait()


def kernel(modality_embeddings, modality_idx):
    idx = jnp.asarray(modality_idx, dtype=jnp.int32).reshape((1,))
    return pl.pallas_call(
        _row_dma,
        in_specs=[
            pl.BlockSpec(memory_space=pltpu.SMEM),
            pl.BlockSpec(memory_space=pl.ANY),
        ],
        out_specs=pl.BlockSpec(memory_space=pl.ANY),
        out_shape=jax.ShapeDtypeStruct((_EMBED_DIM,),
                                       modality_embeddings.dtype),
        scratch_shapes=[pltpu.SemaphoreType.DMA],
    )(idx, modality_embeddings)
